# baseline (speedup 1.0000x reference)
import functools

import jax
import jax.numpy as jnp
from jax import lax
from jax.experimental import pallas as pl
from jax.experimental.pallas import tpu as pltpu
from jax.experimental.pallas import tpu_sc as plsc

NUM_EMB = 1000000
EMBEDDING_DIM = 32


@functools.partial(jax.jit, static_argnums=(0, 1, 2))
def _gather_call(I, J, S, idx_flat, table):
    D = EMBEDDING_DIM
    K = S * J
    info = plsc.get_sparse_core_info()
    NW = info.num_cores * info.num_subcores
    span_s = I // NW
    NC = span_s // S
    assert I % NW == 0 and span_s % S == 0 and NC % 2 == 0
    mesh = plsc.VectorSubcoreMesh(core_axis_name="c", subcore_axis_name="s")

    @functools.partial(
        pl.kernel,
        mesh=mesh,
        out_type=jax.ShapeDtypeStruct((I, J, D), jnp.float32),
        scratch_types=[
            pltpu.VMEM((2, K), jnp.int32),
            pltpu.VMEM((2, K, D), jnp.float32),
            pltpu.SemaphoreType.DMA,
            pltpu.SemaphoreType.DMA,
        ],
        compiler_params=pltpu.CompilerParams(
            use_tc_tiling_on_sc=False, needs_layout_passes=False
        ),
    )
    def k(idx_hbm, table_hbm, out_hbm, idx_v, blk_v, gsem0, gsem1):
        gsems = (gsem0, gsem1)
        wid = lax.axis_index("s") * info.num_cores + lax.axis_index("c")
        base = wid * span_s

        def start(c, b):
            pltpu.sync_copy(
                idx_hbm.at[pl.ds((base + c * S) * J, K)], idx_v.at[b]
            )
            pltpu.async_copy(table_hbm.at[idx_v.at[b]], blk_v.at[b], gsems[b])

        start(0, 0)
        start(1, 1)

        def body(n, carry):
            c0 = n * 2
            for b in range(2):
                c = c0 + b
                pltpu.make_async_copy(
                    table_hbm.at[idx_v.at[b]], blk_v.at[b], gsems[b]
                ).wait()
                for s in range(S):
                    pltpu.sync_copy(
                        blk_v.at[b, pl.ds(s * J, J)],
                        out_hbm.at[base + c * S + s],
                    )

                @pl.when(c + 2 < NC)
                def _():
                    start(c + 2, b)

            return carry

        lax.fori_loop(0, NC // 2, body, 0)

    return k(idx_flat, table)


def kernel(input, table):
    I, J = input.shape
    idx_flat = input.ravel().astype(jnp.int32)
    return _gather_call(I, J, 8, idx_flat, table)


# speedup vs baseline: 1.1203x; 1.1203x over previous
import functools

import jax
import jax.numpy as jnp
from jax import lax
from jax.experimental import pallas as pl
from jax.experimental.pallas import tpu as pltpu
from jax.experimental.pallas import tpu_sc as plsc

NUM_EMB = 1000000
EMBEDDING_DIM = 32
_PITCH = 524


@functools.partial(jax.jit, static_argnums=(0, 1, 2))
def _gather_call(J, I, C, idxT, table):
    D = EMBEDDING_DIM
    info = plsc.get_sparse_core_info()
    NC, NS = info.num_cores, info.num_subcores
    NW = NC * NS
    assert I == C * NW and J % 2 == 0
    mesh = plsc.VectorSubcoreMesh(core_axis_name="c", subcore_axis_name="s")

    @functools.partial(
        pl.kernel,
        mesh=mesh,
        out_type=jax.ShapeDtypeStruct((J, D, I), jnp.float32),
        scratch_types=[
            pltpu.VMEM((2, C), jnp.int32),
            pltpu.VMEM((2, C, D), jnp.float32),
            pltpu.VMEM((2, D, _PITCH), jnp.float32),
            pltpu.SemaphoreType.DMA,
            pltpu.SemaphoreType.DMA,
        ],
        compiler_params=pltpu.CompilerParams(
            use_tc_tiling_on_sc=False, needs_layout_passes=False
        ),
    )
    def k(idx_hbm, table_hbm, out_hbm, idx_v, blk_v, out_s, gsem0, gsem1):
        gsems = (gsem0, gsem1)
        wid = lax.axis_index("s") * NC + lax.axis_index("c")
        i0 = wid * C

        def start(j, b):
            pltpu.sync_copy(idx_hbm.at[j, pl.ds(i0, C)], idx_v.at[b])
            pltpu.async_copy(table_hbm.at[idx_v.at[b]], blk_v.at[b], gsems[b])

        start(0, 0)
        start(1, 1)

        d_lo = lax.iota(jnp.int32, 16)
        d_hi = d_lo + 16

        def body(n, carry):
            j0 = n * 2
            for b in range(2):
                j = j0 + b
                pltpu.make_async_copy(
                    table_hbm.at[idx_v.at[b]], blk_v.at[b], gsems[b]
                ).wait()

                def nloop(g, c):
                    n0 = g * 8
                    for s in range(8):
                        nn = n0 + s
                        nvec = jnp.full((16,), nn, jnp.int32)
                        row_lo = blk_v[b, nn, pl.ds(0, 16)]
                        row_hi = blk_v[b, nn, pl.ds(16, 16)]
                        plsc.store_scatter(out_s.at[b], [d_lo, nvec], row_lo)
                        plsc.store_scatter(out_s.at[b], [d_hi, nvec], row_hi)
                    return c

                lax.fori_loop(0, C // 8, nloop, 0)
                pltpu.sync_copy(
                    out_s.at[b, :, pl.ds(0, C)], out_hbm.at[j, :, pl.ds(i0, C)]
                )

                @pl.when(j + 2 < J)
                def _():
                    start(j + 2, b)

            return carry

        lax.fori_loop(0, J // 2, body, 0)

    return k(idxT, table)


def kernel(input, table):
    I, J = input.shape
    idxT = input.T.astype(jnp.int32)
    outT = _gather_call(J, I, I // 32, idxT, table)
    return jnp.transpose(outT, (2, 0, 1))
